# baseline (speedup 1.0000x reference)
import functools

import jax
import jax.numpy as jnp
from jax.experimental import pallas as pl
from jax.experimental.pallas import tpu as pltpu

L = 2048
B = 2
D = 768
d = 128
NM = 8
TOPK = 2
CHUNK = 128
F32 = jnp.float32


def _mom_kernel(x_ref, m0_ref, wq_ref, bq_ref, wk_ref, bk_ref, wv_ref,
                bv_ref, wg_ref, bg_ref, out_ref, q_scr, w_scr, k_scr, v_scr,
                ms_scr):
    b = pl.program_id(0)

    def _setup():
        xb = x_ref[...]
        x16 = xb.astype(jnp.bfloat16)
        gt = jax.lax.dot_general(wg_ref[...], xb,
                                 (((1,), (1,)), ((), ())),
                                 preferred_element_type=F32) + bg_ref[...]
        gmax = jnp.max(gt, axis=0, keepdims=True)
        e = jnp.exp(gt - gmax)
        s = e / jnp.sum(e, axis=0, keepdims=True)
        v1 = jnp.max(s, axis=0, keepdims=True)
        c1 = jnp.sum(jnp.where(s == v1, 1.0, 0.0), axis=0, keepdims=True)
        m2 = jnp.max(jnp.where(s == v1, -jnp.inf, s), axis=0, keepdims=True)
        v2 = jnp.where(c1 >= 2.0, v1, m2)
        sel = s >= v2
        alpha = s / (v1 + v2)
        w8t = jnp.where(sel, alpha, -1.0)
        w16t = jnp.concatenate(
            [jnp.ones((1, L), dtype=F32), w8t,
             jnp.full((16 - 1 - NM, L), -1.0, dtype=F32)], axis=0)
        w_scr[...] = w16t.T
        q_scr[...] = (jax.lax.dot_general(
            xb, wq_ref[...], (((1,), (1,)), ((), ())),
            preferred_element_type=F32) + bq_ref[...]).astype(jnp.bfloat16)
        k_scr[...] = (jax.lax.dot_general(
            x16, wk_ref[...], (((1,), (1,)), ((), ())),
            preferred_element_type=F32) + bk_ref[...]).astype(jnp.bfloat16)
        v_scr[...] = (jax.lax.dot_general(
            x16, wv_ref[...], (((1,), (1,)), ((), ())),
            preferred_element_type=F32) + bv_ref[...]).astype(jnp.bfloat16)

    _setup()

    m0 = m0_ref[...]
    row_i = jax.lax.broadcasted_iota(jnp.int32, (CHUNK, CHUNK), 0)
    col_i = jax.lax.broadcasted_iota(jnp.int32, (CHUNK, CHUNK), 1)
    causal = row_i >= col_i
    NC = L // CHUNK
    Mp1 = NM + 1

    lane16 = jax.lax.broadcasted_iota(jnp.int32, (L, 16), 1)

    for m in range(Mp1):
        wrow = jnp.sum(jnp.where(lane16 == m, w_scr[...], 0.0), axis=1,
                       keepdims=True)
        k_scr[:, m * d:(m + 1) * d] = jnp.where(
            wrow >= 0.0, k_scr[:, m * d:(m + 1) * d], jnp.bfloat16(0.0))

    for m in range(Mp1):
        acc = m0
        for c in range(NC):
            ms_scr[c * d:(c + 1) * d, m * d:(m + 1) * d] = (
                acc.astype(jnp.bfloat16))
            if c + 1 < NC:
                inc = jax.lax.dot_general(
                    k_scr[pl.ds(c * CHUNK, CHUNK), pl.ds(m * d, d)],
                    v_scr[pl.ds(c * CHUNK, CHUNK), pl.ds(m * d, d)],
                    (((0,), (0,)), ((), ())), preferred_element_type=F32)
                acc = acc + inc

    for c in range(NC):
        t0 = c * CHUNK
        q = q_scr[pl.ds(t0, CHUNK), :]
        w16 = w_scr[pl.ds(t0, CHUNK), :]
        qstate = jax.lax.dot_general(
            q, ms_scr[c * d:(c + 1) * d, :], (((1,), (0,)), ((), ())),
            preferred_element_type=F32)
        clane = jax.lax.broadcasted_iota(jnp.int32, (CHUNK, 16), 1)
        out_c = None
        for m in range(Mp1):
            wrow = jnp.sum(jnp.where(clane == m, w16, 0.0), axis=1,
                           keepdims=True)
            a = jax.lax.dot_general(
                q, k_scr[pl.ds(t0, CHUNK), pl.ds(m * d, d)],
                (((1,), (1,)), ((), ())),
                preferred_element_type=F32)
            a = jnp.where(causal, a, 0.0).astype(jnp.bfloat16)
            o = (qstate[:, m * d:(m + 1) * d]
                 + jnp.dot(a, v_scr[pl.ds(t0, CHUNK), pl.ds(m * d, d)],
                           preferred_element_type=F32))
            contrib = jnp.maximum(wrow, 0.0) * o
            out_c = contrib if out_c is None else out_c + contrib
        out_ref[pl.ds(t0, CHUNK), :] = out_c


@functools.partial(jax.jit, static_argnames=("interpret",))
def kernel(X, M0, W_q, b_q, W_k, b_k, W_v, b_v, W_g, b_g, interpret=False):
    Mp1 = NM + 1
    x2 = X.reshape(L, B * D)
    wk_bf = W_k.astype(jnp.bfloat16)
    wv_bf = W_v.astype(jnp.bfloat16)
    bq2 = b_q.reshape(1, d)
    bk2 = b_k.reshape(1, Mp1 * d)
    bv2 = b_v.reshape(1, Mp1 * d)
    bg2 = b_g.reshape(NM, 1)

    out = pl.pallas_call(
        _mom_kernel,
        grid=(B,),
        in_specs=[
            pl.BlockSpec((L, D), lambda b: (0, b)),
            pl.BlockSpec((d, d), lambda b: (0, 0)),
            pl.BlockSpec((d, D), lambda b: (0, 0)),
            pl.BlockSpec((1, d), lambda b: (0, 0)),
            pl.BlockSpec((d * Mp1, D), lambda b: (0, 0)),
            pl.BlockSpec((1, Mp1 * d), lambda b: (0, 0)),
            pl.BlockSpec((d * Mp1, D), lambda b: (0, 0)),
            pl.BlockSpec((1, Mp1 * d), lambda b: (0, 0)),
            pl.BlockSpec((NM, D), lambda b: (0, 0)),
            pl.BlockSpec((NM, 1), lambda b: (0, 0)),
        ],
        out_specs=pl.BlockSpec((L, d), lambda b: (0, b)),
        out_shape=jax.ShapeDtypeStruct((L, B * d), F32),
        scratch_shapes=[
            pltpu.VMEM((L, d), jnp.bfloat16),
            pltpu.VMEM((L, 16), F32),
            pltpu.VMEM((L, Mp1 * d), jnp.bfloat16),
            pltpu.VMEM((L, Mp1 * d), jnp.bfloat16),
            pltpu.VMEM(((L // CHUNK) * d, Mp1 * d), jnp.bfloat16),
        ],
        compiler_params=pltpu.CompilerParams(
            dimension_semantics=("parallel",),
        ),
        interpret=interpret,
    )(x2, M0, W_q, bq2, wk_bf, bk2, wv_bf, bv2, W_g, bg2)

    return out.reshape(L, B, d)


# speedup vs baseline: 1.2482x; 1.2482x over previous
import functools

import jax
import jax.numpy as jnp
from jax.experimental import pallas as pl
from jax.experimental.pallas import tpu as pltpu

L = 2048
B = 2
D = 768
d = 128
NM = 8
TOPK = 2
CHUNK = 256
F32 = jnp.float32
BF16 = jnp.bfloat16
Mp1 = NM + 1
NC = L // CHUNK


def _mom_kernel(x_ref, m0_ref, wq_ref, bq_ref, wk_ref, bk_ref, wv_ref,
                bv_ref, wg_ref, bg_ref, out_ref, q_scr, w_scr, k_scr, v_scr,
                ms_scr):
    KL = Mp1 * d

    def _setup(b):
        xb = x_ref[:, b * D:(b + 1) * D]
        x16 = xb.astype(BF16)
        gt = jax.lax.dot_general(wg_ref[...], xb,
                                 (((1,), (1,)), ((), ())),
                                 preferred_element_type=F32) + bg_ref[...]
        gmax = jnp.max(gt, axis=0, keepdims=True)
        e = jnp.exp(gt - gmax)
        s = e / jnp.sum(e, axis=0, keepdims=True)
        v1 = jnp.max(s, axis=0, keepdims=True)
        c1 = jnp.sum(jnp.where(s == v1, 1.0, 0.0), axis=0, keepdims=True)
        m2 = jnp.max(jnp.where(s == v1, -jnp.inf, s), axis=0, keepdims=True)
        v2 = jnp.where(c1 >= 2.0, v1, m2)
        sel = s >= v2
        alpha = s / (v1 + v2)
        w8t = jnp.where(sel, alpha, -1.0)
        w16t = jnp.concatenate(
            [jnp.ones((1, L), dtype=F32), w8t,
             jnp.full((16 - 1 - NM, L), -1.0, dtype=F32)], axis=0)
        w_scr[:, b * 16:(b + 1) * 16] = w16t.T
        q_scr[:, b * d:(b + 1) * d] = (jax.lax.dot_general(
            xb, wq_ref[...], (((1,), (1,)), ((), ())),
            preferred_element_type=F32) + bq_ref[...]).astype(BF16)
        k_scr[:, b * KL:(b + 1) * KL] = (jax.lax.dot_general(
            x16, wk_ref[...], (((1,), (1,)), ((), ())),
            preferred_element_type=F32) + bk_ref[...]).astype(BF16)
        v_scr[:, b * KL:(b + 1) * KL] = (jax.lax.dot_general(
            x16, wv_ref[...], (((1,), (1,)), ((), ())),
            preferred_element_type=F32) + bv_ref[...]).astype(BF16)

    for b in range(B):
        _setup(b)

    m0 = m0_ref[...]
    row_i = jax.lax.broadcasted_iota(jnp.int32, (CHUNK, CHUNK), 0)
    col_i = jax.lax.broadcasted_iota(jnp.int32, (CHUNK, CHUNK), 1)
    causal = row_i >= col_i
    lane16 = jax.lax.broadcasted_iota(jnp.int32, (L, 16), 1)

    for b in range(B):
        w16b = w_scr[:, b * 16:(b + 1) * 16]
        for m in range(Mp1):
            off = b * KL + m * d
            wrow = jnp.sum(jnp.where(lane16 == m, w16b, 0.0), axis=1,
                           keepdims=True)
            k_scr[:, off:off + d] = jnp.where(
                wrow >= 0.0, k_scr[:, off:off + d], BF16(0.0))

    for b in range(B):
        for m in range(Mp1):
            off = b * KL + m * d
            acc = m0
            for c in range(NC):
                ms_scr[c * d:(c + 1) * d, off:off + d] = acc.astype(BF16)
                if c + 1 < NC:
                    inc = jax.lax.dot_general(
                        k_scr[pl.ds(c * CHUNK, CHUNK), pl.ds(off, d)],
                        v_scr[pl.ds(c * CHUNK, CHUNK), pl.ds(off, d)],
                        (((0,), (0,)), ((), ())),
                        preferred_element_type=F32)
                    acc = acc + inc

    clane = jax.lax.broadcasted_iota(jnp.int32, (CHUNK, 16), 1)
    for b in range(B):
        for c in range(NC):
            t0 = c * CHUNK
            q = q_scr[pl.ds(t0, CHUNK), pl.ds(b * d, d)]
            w16 = w_scr[pl.ds(t0, CHUNK), pl.ds(b * 16, 16)]
            qstate = jax.lax.dot_general(
                q, ms_scr[c * d:(c + 1) * d, b * KL:(b + 1) * KL],
                (((1,), (0,)), ((), ())),
                preferred_element_type=F32)
            out_c = None
            for m in range(Mp1):
                off = b * KL + m * d
                wrow = jnp.sum(jnp.where(clane == m, w16, 0.0), axis=1,
                               keepdims=True)
                a = jax.lax.dot_general(
                    q, k_scr[pl.ds(t0, CHUNK), pl.ds(off, d)],
                    (((1,), (1,)), ((), ())),
                    preferred_element_type=F32)
                a = jnp.where(causal, a, 0.0).astype(BF16)
                o = (qstate[:, m * d:(m + 1) * d]
                     + jnp.dot(a, v_scr[pl.ds(t0, CHUNK), pl.ds(off, d)],
                               preferred_element_type=F32))
                contrib = jnp.maximum(wrow, 0.0) * o
                out_c = contrib if out_c is None else out_c + contrib
            out_ref[pl.ds(t0, CHUNK), pl.ds(b * d, d)] = out_c


@functools.partial(jax.jit, static_argnames=("interpret",))
def kernel(X, M0, W_q, b_q, W_k, b_k, W_v, b_v, W_g, b_g, interpret=False):
    wk_bf = W_k.astype(BF16).reshape(Mp1 * d, D)
    wv_bf = W_v.astype(BF16).reshape(Mp1 * d, D)
    bq2 = b_q.reshape(1, d)
    bk2 = b_k.reshape(1, Mp1 * d)
    bv2 = b_v.reshape(1, Mp1 * d)
    bg2 = b_g.reshape(NM, 1)

    x2 = X.reshape(L, B * D)
    out = pl.pallas_call(
        _mom_kernel,
        out_shape=jax.ShapeDtypeStruct((L, B * d), F32),
        scratch_shapes=[
            pltpu.VMEM((L, B * d), BF16),
            pltpu.VMEM((L, B * 16), F32),
            pltpu.VMEM((L, B * Mp1 * d), BF16),
            pltpu.VMEM((L, B * Mp1 * d), BF16),
            pltpu.VMEM((NC * d, B * Mp1 * d), BF16),
        ],
        interpret=interpret,
    )(x2, M0, W_q, bq2, wk_bf, bk2, wv_bf, bv2, W_g, bg2)

    return out.reshape(L, B, d)


# speedup vs baseline: 1.3346x; 1.0692x over previous
import functools

import jax
import jax.numpy as jnp
from jax.experimental import pallas as pl
from jax.experimental.pallas import tpu as pltpu

L = 2048
B = 2
D = 768
d = 128
NM = 8
TOPK = 2
CHUNK = 256
F32 = jnp.float32


def _mom_kernel(x_ref, m0_ref, wq_ref, bq_ref, wk_ref, bk_ref, wv_ref,
                bv_ref, wg_ref, bg_ref, out_ref, q_scr, w_scr, k_scr, v_scr,
                ms_scr):
    b = pl.program_id(0)

    def _setup():
        xb = x_ref[...]
        x16 = xb.astype(jnp.bfloat16)
        gt = jax.lax.dot_general(wg_ref[...], xb,
                                 (((1,), (1,)), ((), ())),
                                 preferred_element_type=F32) + bg_ref[...]
        gmax = jnp.max(gt, axis=0, keepdims=True)
        e = jnp.exp(gt - gmax)
        s = e / jnp.sum(e, axis=0, keepdims=True)
        v1 = jnp.max(s, axis=0, keepdims=True)
        c1 = jnp.sum(jnp.where(s == v1, 1.0, 0.0), axis=0, keepdims=True)
        m2 = jnp.max(jnp.where(s == v1, -jnp.inf, s), axis=0, keepdims=True)
        v2 = jnp.where(c1 >= 2.0, v1, m2)
        sel = s >= v2
        alpha = s / (v1 + v2)
        w8t = jnp.where(sel, alpha, -1.0)
        w16t = jnp.concatenate(
            [jnp.ones((1, L), dtype=F32), w8t,
             jnp.full((16 - 1 - NM, L), -1.0, dtype=F32)], axis=0)
        w_scr[...] = w16t.T
        q_scr[...] = (jax.lax.dot_general(
            xb, wq_ref[...], (((1,), (1,)), ((), ())),
            preferred_element_type=F32) + bq_ref[...]).astype(jnp.bfloat16)
        k_scr[...] = (jax.lax.dot_general(
            x16, wk_ref[...], (((1,), (1,)), ((), ())),
            preferred_element_type=F32) + bk_ref[...]).astype(jnp.bfloat16)
        v_scr[...] = (jax.lax.dot_general(
            x16, wv_ref[...], (((1,), (1,)), ((), ())),
            preferred_element_type=F32) + bv_ref[...]).astype(jnp.bfloat16)

    _setup()

    m0 = m0_ref[...]
    row_i = jax.lax.broadcasted_iota(jnp.int32, (CHUNK, CHUNK), 0)
    col_i = jax.lax.broadcasted_iota(jnp.int32, (CHUNK, CHUNK), 1)
    causal = row_i >= col_i
    NC = L // CHUNK
    Mp1 = NM + 1

    lane16 = jax.lax.broadcasted_iota(jnp.int32, (L, 16), 1)

    for m in range(Mp1):
        wrow = jnp.sum(jnp.where(lane16 == m, w_scr[...], 0.0), axis=1,
                       keepdims=True)
        k_scr[:, m * d:(m + 1) * d] = jnp.where(
            wrow >= 0.0, k_scr[:, m * d:(m + 1) * d], jnp.bfloat16(0.0))

    for m in range(Mp1):
        acc = m0
        for c in range(NC):
            ms_scr[c * d:(c + 1) * d, m * d:(m + 1) * d] = (
                acc.astype(jnp.bfloat16))
            if c + 1 < NC:
                inc = jax.lax.dot_general(
                    k_scr[pl.ds(c * CHUNK, CHUNK), pl.ds(m * d, d)],
                    v_scr[pl.ds(c * CHUNK, CHUNK), pl.ds(m * d, d)],
                    (((0,), (0,)), ((), ())), preferred_element_type=F32)
                acc = acc + inc

    for c in range(NC):
        t0 = c * CHUNK
        q = q_scr[pl.ds(t0, CHUNK), :]
        w16 = w_scr[pl.ds(t0, CHUNK), :]
        qstate = jax.lax.dot_general(
            q, ms_scr[c * d:(c + 1) * d, :], (((1,), (0,)), ((), ())),
            preferred_element_type=F32)
        clane = jax.lax.broadcasted_iota(jnp.int32, (CHUNK, 16), 1)
        out_c = None
        for m in range(Mp1):
            wrow = jnp.sum(jnp.where(clane == m, w16, 0.0), axis=1,
                           keepdims=True)
            a = jax.lax.dot_general(
                q, k_scr[pl.ds(t0, CHUNK), pl.ds(m * d, d)],
                (((1,), (1,)), ((), ())),
                preferred_element_type=F32)
            a = jnp.where(causal, a, 0.0).astype(jnp.bfloat16)
            o = (qstate[:, m * d:(m + 1) * d]
                 + jnp.dot(a, v_scr[pl.ds(t0, CHUNK), pl.ds(m * d, d)],
                           preferred_element_type=F32))
            contrib = jnp.maximum(wrow, 0.0) * o
            out_c = contrib if out_c is None else out_c + contrib
        out_ref[pl.ds(t0, CHUNK), :] = out_c


@functools.partial(jax.jit, static_argnames=("interpret",))
def kernel(X, M0, W_q, b_q, W_k, b_k, W_v, b_v, W_g, b_g, interpret=False):
    Mp1 = NM + 1
    x2 = X.reshape(L, B * D)
    wk_bf = W_k.astype(jnp.bfloat16)
    wv_bf = W_v.astype(jnp.bfloat16)
    bq2 = b_q.reshape(1, d)
    bk2 = b_k.reshape(1, Mp1 * d)
    bv2 = b_v.reshape(1, Mp1 * d)
    bg2 = b_g.reshape(NM, 1)

    out = pl.pallas_call(
        _mom_kernel,
        grid=(B,),
        in_specs=[
            pl.BlockSpec((L, D), lambda b: (0, b)),
            pl.BlockSpec((d, d), lambda b: (0, 0)),
            pl.BlockSpec((d, D), lambda b: (0, 0)),
            pl.BlockSpec((1, d), lambda b: (0, 0)),
            pl.BlockSpec((d * Mp1, D), lambda b: (0, 0)),
            pl.BlockSpec((1, Mp1 * d), lambda b: (0, 0)),
            pl.BlockSpec((d * Mp1, D), lambda b: (0, 0)),
            pl.BlockSpec((1, Mp1 * d), lambda b: (0, 0)),
            pl.BlockSpec((NM, D), lambda b: (0, 0)),
            pl.BlockSpec((NM, 1), lambda b: (0, 0)),
        ],
        out_specs=pl.BlockSpec((L, d), lambda b: (0, b)),
        out_shape=jax.ShapeDtypeStruct((L, B * d), F32),
        scratch_shapes=[
            pltpu.VMEM((L, d), jnp.bfloat16),
            pltpu.VMEM((L, 16), F32),
            pltpu.VMEM((L, Mp1 * d), jnp.bfloat16),
            pltpu.VMEM((L, Mp1 * d), jnp.bfloat16),
            pltpu.VMEM(((L // CHUNK) * d, Mp1 * d), jnp.bfloat16),
        ],
        compiler_params=pltpu.CompilerParams(
            dimension_semantics=("parallel",),
        ),
        interpret=interpret,
    )(x2, M0, W_q, bq2, wk_bf, bk2, wv_bf, bv2, W_g, bg2)

    return out.reshape(L, B, d)
